# baseline (device time: 30371 ns/iter reference)
import numpy as np
import jax
import jax.numpy as jnp
from jax import lax
from jax.experimental import pallas as pl
from jax.experimental.pallas import tpu as pltpu

N_DEV = 4
B, SQ, D = 2, 256, 768
H_LOC, DH = 4, 64
HD = H_LOC * DH


def _rope_consts():
    inv = 1.0 / (10000.0 ** (np.arange(0, DH, 2) / DH))
    pos = np.arange(SQ)[:, None] * inv[None, :]
    cos = np.repeat(np.cos(pos), 2, axis=-1).astype(np.float32)
    sin = np.repeat(np.sin(pos), 2, axis=-1).astype(np.float32)
    R = np.zeros((DH, DH), dtype=np.float32)
    for k in range(DH // 2):
        R[2 * k + 1, 2 * k] = -1.0
        R[2 * k, 2 * k + 1] = 1.0
    return cos, sin, R


def kernel(x, Wq, Wk, Wv, Wo):
    cos_np, sin_np, R_np = _rope_consts()
    cos_c = jnp.asarray(cos_np)
    sin_c = jnp.asarray(sin_np)
    R_c = jnp.asarray(R_np)

    def body(x_ref, wq_ref, wk_ref, wv_ref, wo_ref, cos_ref, sin_ref, r_ref,
             out_ref, ctx_ref, wo_src, ctx_full, wo_full,
             wo_ssems, wo_rsems, ctx_ssems, ctx_rsems):
        my = lax.axis_index("i")

        f32 = jnp.float32
        bf16 = jnp.bfloat16

        barrier_sem = pltpu.get_barrier_semaphore()
        for d in range(1, N_DEV):
            pl.semaphore_signal(
                barrier_sem, inc=1,
                device_id=(lax.rem(my + d, N_DEV),),
                device_id_type=pl.DeviceIdType.MESH,
            )
        pl.semaphore_wait(barrier_sem, N_DEV - 1)

        sends = []

        wo_src[...] = wo_ref[...].astype(bf16)
        wo_full[my] = wo_src[...]
        for d in range(1, N_DEV):
            t = lax.rem(my + d, N_DEV)
            rdma = pltpu.make_async_remote_copy(
                src_ref=wo_src,
                dst_ref=wo_full.at[my],
                send_sem=wo_ssems.at[d - 1],
                recv_sem=wo_rsems.at[my],
                device_id=(t,),
                device_id_type=pl.DeviceIdType.MESH,
            )
            rdma.start()
            sends.append(rdma)

        cos = cos_ref[...]
        sin = sin_ref[...]
        R = r_ref[...]
        wq = wq_ref[...].astype(bf16)
        wk = wk_ref[...].astype(bf16)
        wv = wv_ref[...].astype(bf16)

        for b in range(B):
            xb = x_ref[b].astype(bf16)
            q = jnp.dot(xb, wq, preferred_element_type=f32)
            k = jnp.dot(xb, wk, preferred_element_type=f32)
            v = jnp.dot(xb, wv, preferred_element_type=f32)
            for h in range(H_LOC):
                sl = slice(h * DH, (h + 1) * DH)
                qh = q[:, sl]
                kh = k[:, sl]
                qh = qh * cos + jnp.dot(qh, R, preferred_element_type=f32) * sin
                kh = kh * cos + jnp.dot(kh, R, preferred_element_type=f32) * sin
                s = lax.dot_general(
                    qh.astype(bf16), kh.astype(bf16),
                    (((1,), (1,)), ((), ())),
                    preferred_element_type=f32,
                ) * 0.125
                s = s - jnp.max(s, axis=-1, keepdims=True)
                w = jnp.exp(s)
                w = w / jnp.sum(w, axis=-1, keepdims=True)
                ctx = jnp.dot(
                    w.astype(bf16), v[:, sl].astype(bf16),
                    preferred_element_type=f32,
                )
                ctx_ref[b, :, sl] = ctx.astype(bf16)
            ctx_full[my, b] = ctx_ref[b]
            for d in range(1, N_DEV):
                t = lax.rem(my + d, N_DEV)
                rdma = pltpu.make_async_remote_copy(
                    src_ref=ctx_ref.at[b],
                    dst_ref=ctx_full.at[my, b],
                    send_sem=ctx_ssems.at[(d - 1) * B + b],
                    recv_sem=ctx_rsems.at[my, b],
                    device_id=(t,),
                    device_id_type=pl.DeviceIdType.MESH,
                )
                rdma.start()
                sends.append(rdma)

        for b in range(B):
            out_ref[b] = jnp.dot(ctx_ref[b], wo_src[...],
                                 preferred_element_type=f32)

        for d in range(1, N_DEV):
            m = lax.rem(my + d, N_DEV)
            wo_recv = pltpu.make_async_remote_copy(
                src_ref=wo_full.at[m],
                dst_ref=wo_full.at[m],
                send_sem=wo_ssems.at[d - 1],
                recv_sem=wo_rsems.at[m],
                device_id=(my,),
                device_id_type=pl.DeviceIdType.MESH,
            )
            wo_recv.wait_recv()
            wom = wo_full[m]
            for b in range(B):
                ctx_recv = pltpu.make_async_remote_copy(
                    src_ref=ctx_full.at[m, b],
                    dst_ref=ctx_full.at[m, b],
                    send_sem=ctx_ssems.at[(d - 1) * B + b],
                    recv_sem=ctx_rsems.at[m, b],
                    device_id=(my,),
                    device_id_type=pl.DeviceIdType.MESH,
                )
                ctx_recv.wait_recv()
                out_ref[b] = out_ref[b] + jnp.dot(
                    ctx_full[m, b], wom, preferred_element_type=f32
                )

        for rdma in sends:
            rdma.wait_send()

    return pl.pallas_call(
        body,
        out_shape=jax.ShapeDtypeStruct((B, SQ, D), jnp.float32),
        in_specs=[pl.BlockSpec(memory_space=pltpu.VMEM)] * 8,
        out_specs=pl.BlockSpec(memory_space=pltpu.VMEM),
        scratch_shapes=[
            pltpu.VMEM((B, SQ, HD), jnp.bfloat16),
            pltpu.VMEM((HD, D), jnp.bfloat16),
            pltpu.VMEM((N_DEV, B, SQ, HD), jnp.bfloat16),
            pltpu.VMEM((N_DEV, HD, D), jnp.bfloat16),
            pltpu.SemaphoreType.DMA((N_DEV - 1,)),
            pltpu.SemaphoreType.DMA((N_DEV,)),
            pltpu.SemaphoreType.DMA(((N_DEV - 1) * B,)),
            pltpu.SemaphoreType.DMA((N_DEV, B)),
        ],
        compiler_params=pltpu.CompilerParams(collective_id=0),
    )(x, Wq, Wk, Wv, Wo, cos_c, sin_c, R_c)


# device time: 20583 ns/iter; 1.4755x vs baseline; 1.4755x over previous
import os

import numpy as np
import jax
import jax.numpy as jnp
from jax import lax
from jax.experimental import pallas as pl
from jax.experimental.pallas import tpu as pltpu

N_DEV = 4
_ABL = int(os.environ.get("ABL", "0"))
B, SQ, D = 2, 256, 768
H_LOC, DH = 4, 64
HD = H_LOC * DH


def _rope_consts():
    inv = 1.0 / (10000.0 ** (np.arange(0, DH, 2) / DH))
    pos = np.arange(SQ)[:, None] * inv[None, :]
    cos = np.repeat(np.cos(pos), 2, axis=-1).astype(np.float32)
    sin = np.repeat(np.sin(pos), 2, axis=-1).astype(np.float32)
    R = np.zeros((DH, DH), dtype=np.float32)
    for k in range(DH // 2):
        R[2 * k + 1, 2 * k] = -1.0
        R[2 * k, 2 * k + 1] = 1.0
    return cos, sin, R


def kernel(x, Wq, Wk, Wv, Wo):
    cos_np, sin_np, R_np = _rope_consts()
    cos_c = jnp.asarray(cos_np)
    sin_c = jnp.asarray(sin_np)
    R_c = jnp.asarray(R_np)

    def body(x_ref, wq_ref, wk_ref, wv_ref, wo_ref, cos_ref, sin_ref, r_ref,
             out_ref, ctx_ref, wo_src, ctx_full, wo_full,
             wo_ssems, wo_rsems, ctx_ssems, ctx_rsems):
        my = lax.axis_index("i")

        f32 = jnp.float32
        bf16 = jnp.bfloat16

        barrier_sem = pltpu.get_barrier_semaphore()
        for d in range(1, N_DEV):
            pl.semaphore_signal(
                barrier_sem, inc=1,
                device_id=(lax.rem(my + d, N_DEV),),
                device_id_type=pl.DeviceIdType.MESH,
            )
        pl.semaphore_wait(barrier_sem, N_DEV - 1)

        sends = []

        wo_src[...] = wo_ref[...].astype(bf16)
        wo_full[my] = wo_src[...]
        for d in range(1, N_DEV) if not _ABL else ():
            t = lax.rem(my + d, N_DEV)
            rdma = pltpu.make_async_remote_copy(
                src_ref=wo_src,
                dst_ref=wo_full.at[my],
                send_sem=wo_ssems.at[d - 1],
                recv_sem=wo_rsems.at[my],
                device_id=(t,),
                device_id_type=pl.DeviceIdType.MESH,
            )
            rdma.start()
            sends.append(rdma)

        cos = cos_ref[...]
        sin = sin_ref[...]
        R = r_ref[...]
        wq = wq_ref[...].astype(bf16)
        wk = wk_ref[...].astype(bf16)
        wv = wv_ref[...].astype(bf16)

        for b in range(B):
            xb = x_ref[b].astype(bf16)
            q = jnp.dot(xb, wq, preferred_element_type=f32)
            k = jnp.dot(xb, wk, preferred_element_type=f32)
            v = jnp.dot(xb, wv, preferred_element_type=f32)
            for h in range(H_LOC):
                sl = slice(h * DH, (h + 1) * DH)
                qh = q[:, sl]
                kh = k[:, sl]
                qh = qh * cos + jnp.dot(qh, R, preferred_element_type=f32) * sin
                kh = kh * cos + jnp.dot(kh, R, preferred_element_type=f32) * sin
                s = lax.dot_general(
                    qh.astype(bf16), kh.astype(bf16),
                    (((1,), (1,)), ((), ())),
                    preferred_element_type=f32,
                ) * 0.125
                s = s - jnp.max(s, axis=-1, keepdims=True)
                w = jnp.exp(s)
                w = w / jnp.sum(w, axis=-1, keepdims=True)
                ctx = jnp.dot(
                    w.astype(bf16), v[:, sl].astype(bf16),
                    preferred_element_type=f32,
                )
                ctx_ref[b, :, sl] = ctx.astype(bf16)
            ctx_full[my, b] = ctx_ref[b]
            for d in range(1, N_DEV) if not _ABL else ():
                t = lax.rem(my + d, N_DEV)
                rdma = pltpu.make_async_remote_copy(
                    src_ref=ctx_ref.at[b],
                    dst_ref=ctx_full.at[my, b],
                    send_sem=ctx_ssems.at[(d - 1) * B + b],
                    recv_sem=ctx_rsems.at[my, b],
                    device_id=(t,),
                    device_id_type=pl.DeviceIdType.MESH,
                )
                rdma.start()
                sends.append(rdma)

        for b in range(B):
            out_ref[b] = jnp.dot(ctx_ref[b], wo_src[...],
                                 preferred_element_type=f32)

        for d in range(1, N_DEV) if not _ABL else ():
            m = lax.rem(my + d, N_DEV)
            wo_recv = pltpu.make_async_remote_copy(
                src_ref=wo_full.at[m],
                dst_ref=wo_full.at[m],
                send_sem=wo_ssems.at[d - 1],
                recv_sem=wo_rsems.at[m],
                device_id=(my,),
                device_id_type=pl.DeviceIdType.MESH,
            )
            wo_recv.wait_recv()
            wom = wo_full[m]
            for b in range(B):
                ctx_recv = pltpu.make_async_remote_copy(
                    src_ref=ctx_full.at[m, b],
                    dst_ref=ctx_full.at[m, b],
                    send_sem=ctx_ssems.at[(d - 1) * B + b],
                    recv_sem=ctx_rsems.at[m, b],
                    device_id=(my,),
                    device_id_type=pl.DeviceIdType.MESH,
                )
                ctx_recv.wait_recv()
                out_ref[b] = out_ref[b] + jnp.dot(
                    ctx_full[m, b], wom, preferred_element_type=f32
                )

        for rdma in sends:
            rdma.wait_send()

    return pl.pallas_call(
        body,
        out_shape=jax.ShapeDtypeStruct((B, SQ, D), jnp.float32),
        in_specs=[pl.BlockSpec(memory_space=pltpu.VMEM)] * 8,
        out_specs=pl.BlockSpec(memory_space=pltpu.VMEM),
        scratch_shapes=[
            pltpu.VMEM((B, SQ, HD), jnp.bfloat16),
            pltpu.VMEM((HD, D), jnp.bfloat16),
            pltpu.VMEM((N_DEV, B, SQ, HD), jnp.bfloat16),
            pltpu.VMEM((N_DEV, HD, D), jnp.bfloat16),
            pltpu.SemaphoreType.DMA((N_DEV - 1,)),
            pltpu.SemaphoreType.DMA((N_DEV,)),
            pltpu.SemaphoreType.DMA(((N_DEV - 1) * B,)),
            pltpu.SemaphoreType.DMA((N_DEV, B)),
        ],
        compiler_params=pltpu.CompilerParams(collective_id=0),
    )(x, Wq, Wk, Wv, Wo, cos_c, sin_c, R_c)
